# baseline (speedup 1.0000x reference)
import functools

import jax
import jax.numpy as jnp
from jax import lax
from jax.experimental import pallas as pl
from jax.experimental.pallas import tpu as pltpu
from jax.experimental.pallas import tpu_sc as plsc

NUM_ROWS = 100001
DIM = 32
BATCH = 16384
LANES = 16


def kernel(idx, table):
    info = plsc.get_sparse_core_info()
    num_cores, num_subcores = info.num_cores, info.num_subcores
    num_workers = num_cores * num_subcores
    b_per_w = BATCH // num_workers
    n_groups = b_per_w // LANES

    mesh = plsc.VectorSubcoreMesh(core_axis_name="c", subcore_axis_name="s")

    @functools.partial(
        pl.kernel,
        mesh=mesh,
        out_type=jax.ShapeDtypeStruct((BATCH, DIM), jnp.float32),
        scratch_types=[
            pltpu.VMEM((b_per_w,), jnp.int32),
            pltpu.SemaphoreType.DMA,
        ],
        compiler_params=pltpu.CompilerParams(needs_layout_passes=False),
    )
    def gather_kernel(table_hbm, idx_hbm, out_hbm, idx_v, sem):
        wid = lax.axis_index("s") * num_cores + lax.axis_index("c")
        base = wid * b_per_w
        pltpu.sync_copy(idx_hbm.at[pl.ds(base, b_per_w)], idx_v)
        lanes = lax.broadcasted_iota(jnp.int32, (LANES,), 0)

        def fire_group(g, _):
            v = idx_v[pl.ds(g * LANES, LANES)]
            for lane in range(LANES):
                r = jnp.sum(jnp.where(lanes == lane, v, 0))
                pltpu.async_copy(
                    table_hbm.at[pl.ds(r, 1), :],
                    out_hbm.at[pl.ds(base + g * LANES + lane, 1), :],
                    sem,
                )
            return _

        lax.fori_loop(0, n_groups, fire_group, 0)

        def drain(i, _):
            pltpu.make_async_copy(
                table_hbm.at[pl.ds(0, 1), :],
                out_hbm.at[pl.ds(base + i, 1), :],
                sem,
            ).wait()
            return _

        lax.fori_loop(0, b_per_w, drain, 0)

    return gather_kernel(table, idx)


# speedup vs baseline: 8.7095x; 8.7095x over previous
import functools

import jax
import jax.numpy as jnp
from jax import lax
from jax.experimental import pallas as pl
from jax.experimental.pallas import tpu as pltpu
from jax.experimental.pallas import tpu_sc as plsc

NUM_ROWS = 100001
DIM = 32
BATCH = 16384
LANES = 16
CHUNK = 8192


def kernel(idx, table):
    info = plsc.get_sparse_core_info()
    num_cores, num_subcores = info.num_cores, info.num_subcores
    num_workers = num_cores * num_subcores
    assert num_workers == DIM

    mesh = plsc.VectorSubcoreMesh(core_axis_name="c", subcore_axis_name="s")

    @functools.partial(
        pl.kernel,
        mesh=mesh,
        out_type=jax.ShapeDtypeStruct((DIM, BATCH), jnp.float32),
        scratch_types=[
            pltpu.VMEM((1, NUM_ROWS), jnp.float32),
            pltpu.VMEM((CHUNK,), jnp.int32),
            pltpu.VMEM((1, CHUNK), jnp.float32),
        ],
        compiler_params=pltpu.CompilerParams(needs_layout_passes=False),
    )
    def gather_kernel(tab_t, idx_hbm, out_t, col_v, idx_v, out_v):
        wid = lax.axis_index("s") * num_cores + lax.axis_index("c")
        pltpu.sync_copy(tab_t.at[pl.ds(wid, 1), :], col_v)
        zeros = jnp.zeros((LANES,), jnp.int32)

        for chunk in range(BATCH // CHUNK):
            start = chunk * CHUNK
            pltpu.sync_copy(idx_hbm.at[pl.ds(start, CHUNK)], idx_v)

            def gather_group(g, _):
                iv = idx_v[pl.ds(g * LANES, LANES)]
                vals = plsc.load_gather(col_v, [zeros, iv])
                out_v[0, pl.ds(g * LANES, LANES)] = vals
                return _

            lax.fori_loop(0, CHUNK // LANES, gather_group, 0)
            pltpu.sync_copy(out_v, out_t.at[pl.ds(wid, 1), pl.ds(start, CHUNK)])

    return gather_kernel(table.T, idx).T
